# baseline (speedup 1.0000x reference)
import functools

import jax
import jax.numpy as jnp
from jax import lax
from jax.experimental import pallas as pl
from jax.experimental.pallas import tpu as pltpu
from jax.experimental.pallas import tpu_sc as plsc

EMB = 512
HID = 1024
EPS = 1e-07

_NC = 2
_NS = 16
_NW = _NC * _NS

_CHUNK = 64


def _sc_gather_body(ids_hbm, table_hbm, out_hbm, idx_v, buf_v, sem):
    n_tok = ids_hbm.shape[0]
    tok_per_w = n_tok // _NW
    n_chunks = tok_per_w // _CHUNK
    wid = lax.axis_index("s") * _NC + lax.axis_index("c")
    base = wid * tok_per_w
    for c in range(n_chunks):
        off = base + c * _CHUNK
        pltpu.sync_copy(ids_hbm.at[pl.ds(off, _CHUNK)], idx_v)
        pltpu.async_copy(table_hbm.at[idx_v], buf_v, sem).wait()
        pltpu.sync_copy(buf_v, out_hbm.at[pl.ds(off, _CHUNK)])


def _sc_gather(ids_flat, word_table):
    n_tok = ids_flat.shape[0]
    mesh = plsc.VectorSubcoreMesh(core_axis_name="c", subcore_axis_name="s")
    k = functools.partial(
        pl.kernel,
        mesh=mesh,
        out_type=jax.ShapeDtypeStruct((n_tok, EMB), jnp.float32),
        scratch_types=[
            pltpu.VMEM((_CHUNK,), jnp.int32),
            pltpu.VMEM((_CHUNK, EMB), jnp.float32),
            pltpu.SemaphoreType.DMA,
        ],
    )(_sc_gather_body)
    return k(ids_flat, word_table)


def _tc_body(x_ref, pos_ref, w_ref, g_ref, b_ref, o_ref):
    x = (x_ref[...] + pos_ref[...]).astype(jnp.bfloat16)
    h = jnp.dot(x, w_ref[...], preferred_element_type=jnp.float32)
    mu = jnp.mean(h, axis=-1, keepdims=True)
    var = jnp.mean((h - mu) ** 2, axis=-1, keepdims=True)
    o_ref[...] = (h - mu) * lax.rsqrt(var + EPS) * g_ref[...] + b_ref[...]


def _tc_proj_ln(embeds, pos_table, proj_w, ln_gamma, ln_beta, seq_len):
    n_tok = embeds.shape[0]
    blk = 256
    grid = (n_tok // blk,)
    pos_blocks = seq_len // blk
    out = pl.pallas_call(
        _tc_body,
        grid=grid,
        in_specs=[
            pl.BlockSpec((blk, EMB), lambda i: (i, 0)),
            pl.BlockSpec((blk, EMB), lambda i: (i % pos_blocks, 0)),
            pl.BlockSpec((EMB, HID), lambda i: (0, 0)),
            pl.BlockSpec((1, HID), lambda i: (0, 0)),
            pl.BlockSpec((1, HID), lambda i: (0, 0)),
        ],
        out_specs=pl.BlockSpec((blk, HID), lambda i: (i, 0)),
        out_shape=jax.ShapeDtypeStruct((n_tok, HID), jnp.float32),
    )(embeds, pos_table[:seq_len], proj_w.astype(jnp.bfloat16),
      ln_gamma.reshape(1, HID), ln_beta.reshape(1, HID))
    return out


def kernel(input_ids, word_table, pos_table, proj_w, ln_gamma, ln_beta):
    bsz, seq_len = input_ids.shape
    ids_flat = input_ids.reshape(-1).astype(jnp.int32)
    embeds = _sc_gather(ids_flat, word_table)
    h = _tc_proj_ln(embeds, pos_table, proj_w, ln_gamma, ln_beta, seq_len)
    return h.reshape(bsz, seq_len, HID)


# speedup vs baseline: 1.0360x; 1.0360x over previous
import functools

import jax
import jax.numpy as jnp
from jax import lax
from jax.experimental import pallas as pl
from jax.experimental.pallas import tpu as pltpu
from jax.experimental.pallas import tpu_sc as plsc

EMB = 512
HID = 1024
EPS = 1e-07

_NC = 2
_NS = 16
_NW = _NC * _NS

_CHUNK = 64


def _sc_gather_body(ids_hbm, table_hbm, out_hbm, idx_v, buf_v, sem):
    n_tok = ids_hbm.shape[0]
    tok_per_w = n_tok // _NW
    n_chunks = tok_per_w // _CHUNK
    wid = lax.axis_index("s") * _NC + lax.axis_index("c")
    base = wid * tok_per_w
    for c in range(n_chunks):
        off = base + c * _CHUNK
        pltpu.sync_copy(ids_hbm.at[pl.ds(off, _CHUNK)], idx_v)
        pltpu.async_copy(table_hbm.at[idx_v], buf_v, sem).wait()
        pltpu.sync_copy(buf_v, out_hbm.at[pl.ds(off, _CHUNK)])


def _sc_gather(ids_flat, word_table):
    n_tok = ids_flat.shape[0]
    mesh = plsc.VectorSubcoreMesh(core_axis_name="c", subcore_axis_name="s")
    k = functools.partial(
        pl.kernel,
        mesh=mesh,
        out_type=jax.ShapeDtypeStruct((n_tok, EMB), jnp.float32),
        scratch_types=[
            pltpu.VMEM((_CHUNK,), jnp.int32),
            pltpu.VMEM((_CHUNK, EMB), jnp.float32),
            pltpu.SemaphoreType.DMA,
        ],
    )(_sc_gather_body)
    return k(ids_flat, word_table)


def _tc_body(x_ref, pos_ref, w_ref, g_ref, b_ref, o_ref):
    x = (x_ref[...] + pos_ref[...]).astype(jnp.bfloat16)
    h = jnp.dot(x, w_ref[...], preferred_element_type=jnp.float32)
    mu = jnp.mean(h, axis=-1, keepdims=True)
    var = jnp.mean((h - mu) ** 2, axis=-1, keepdims=True)
    o_ref[...] = (h - mu) * lax.rsqrt(var + EPS) * g_ref[...] + b_ref[...]


def _tc_proj_ln(embeds, pos_table, proj_w, ln_gamma, ln_beta, seq_len):
    n_tok = embeds.shape[0]
    blk = 256
    nbatch = n_tok // seq_len
    pos_blocks = seq_len // blk
    out = pl.pallas_call(
        _tc_body,
        grid=(pos_blocks, nbatch),
        in_specs=[
            pl.BlockSpec((blk, EMB), lambda j, b: (b * pos_blocks + j, 0)),
            pl.BlockSpec((blk, EMB), lambda j, b: (j, 0)),
            pl.BlockSpec((EMB, HID), lambda j, b: (0, 0)),
            pl.BlockSpec((1, HID), lambda j, b: (0, 0)),
            pl.BlockSpec((1, HID), lambda j, b: (0, 0)),
        ],
        out_specs=pl.BlockSpec((blk, HID), lambda j, b: (b * pos_blocks + j, 0)),
        out_shape=jax.ShapeDtypeStruct((n_tok, HID), jnp.float32),
    )(embeds, pos_table[:seq_len], proj_w.astype(jnp.bfloat16),
      ln_gamma.reshape(1, HID), ln_beta.reshape(1, HID))
    return out


def kernel(input_ids, word_table, pos_table, proj_w, ln_gamma, ln_beta):
    bsz, seq_len = input_ids.shape
    ids_flat = input_ids.reshape(-1).astype(jnp.int32)
    embeds = _sc_gather(ids_flat, word_table)
    h = _tc_proj_ln(embeds, pos_table, proj_w, ln_gamma, ln_beta, seq_len)
    return h.reshape(bsz, seq_len, HID)


# speedup vs baseline: 1.0734x; 1.0361x over previous
import functools

import jax
import jax.numpy as jnp
from jax import lax
from jax.experimental import pallas as pl
from jax.experimental.pallas import tpu as pltpu
from jax.experimental.pallas import tpu_sc as plsc

EMB = 512
HID = 1024
EPS = 1e-07

_NC = 2
_NS = 16
_NW = _NC * _NS

_CHUNK = 64


def _sc_gather_body(ids_hbm, table_hbm, out_hbm, idx_v, buf0, buf1, s0, s1):
    n_tok = ids_hbm.shape[0]
    tok_per_w = n_tok // _NW
    n_chunks = tok_per_w // _CHUNK
    wid = lax.axis_index("s") * _NC + lax.axis_index("c")
    base = wid * tok_per_w
    pltpu.sync_copy(ids_hbm.at[pl.ds(base, tok_per_w)], idx_v)
    bufs = (buf0, buf1)
    sems = (s0, s1)
    gathers = [None, None]
    gathers[0] = pltpu.async_copy(
        table_hbm.at[idx_v.at[pl.ds(0, _CHUNK)]], bufs[0], sems[0])
    for c in range(n_chunks):
        if c + 1 < n_chunks:
            gathers[(c + 1) % 2] = pltpu.async_copy(
                table_hbm.at[idx_v.at[pl.ds((c + 1) * _CHUNK, _CHUNK)]],
                bufs[(c + 1) % 2], sems[(c + 1) % 2])
        gathers[c % 2].wait()
        pltpu.sync_copy(bufs[c % 2],
                        out_hbm.at[pl.ds(base + c * _CHUNK, _CHUNK)])


def _sc_gather(ids_flat, word_table):
    n_tok = ids_flat.shape[0]
    tok_per_w = n_tok // _NW
    mesh = plsc.VectorSubcoreMesh(core_axis_name="c", subcore_axis_name="s")
    k = functools.partial(
        pl.kernel,
        mesh=mesh,
        out_type=jax.ShapeDtypeStruct((n_tok, EMB), jnp.float32),
        scratch_types=[
            pltpu.VMEM((tok_per_w,), jnp.int32),
            pltpu.VMEM((_CHUNK, EMB), jnp.float32),
            pltpu.VMEM((_CHUNK, EMB), jnp.float32),
            pltpu.SemaphoreType.DMA,
            pltpu.SemaphoreType.DMA,
        ],
    )(_sc_gather_body)
    return k(ids_flat, word_table)


def _tc_body(x_ref, pos_ref, w_ref, g_ref, b_ref, o_ref):
    x = (x_ref[...] + pos_ref[...]).astype(jnp.bfloat16)
    h = jnp.dot(x, w_ref[...], preferred_element_type=jnp.float32)
    mu = jnp.mean(h, axis=-1, keepdims=True)
    var = jnp.mean((h - mu) ** 2, axis=-1, keepdims=True)
    o_ref[...] = (h - mu) * lax.rsqrt(var + EPS) * g_ref[...] + b_ref[...]


def _tc_proj_ln(embeds, pos_table, proj_w, ln_gamma, ln_beta, seq_len):
    n_tok = embeds.shape[0]
    blk = 256
    nbatch = n_tok // seq_len
    pos_blocks = seq_len // blk
    out = pl.pallas_call(
        _tc_body,
        grid=(pos_blocks, nbatch),
        in_specs=[
            pl.BlockSpec((blk, EMB), lambda j, b: (b * pos_blocks + j, 0)),
            pl.BlockSpec((blk, EMB), lambda j, b: (j, 0)),
            pl.BlockSpec((EMB, HID), lambda j, b: (0, 0)),
            pl.BlockSpec((1, HID), lambda j, b: (0, 0)),
            pl.BlockSpec((1, HID), lambda j, b: (0, 0)),
        ],
        out_specs=pl.BlockSpec((blk, HID), lambda j, b: (b * pos_blocks + j, 0)),
        out_shape=jax.ShapeDtypeStruct((n_tok, HID), jnp.float32),
    )(embeds, pos_table[:seq_len], proj_w.astype(jnp.bfloat16),
      ln_gamma.reshape(1, HID), ln_beta.reshape(1, HID))
    return out


def kernel(input_ids, word_table, pos_table, proj_w, ln_gamma, ln_beta):
    bsz, seq_len = input_ids.shape
    ids_flat = input_ids.reshape(-1).astype(jnp.int32)
    embeds = _sc_gather(ids_flat, word_table)
    h = _tc_proj_ln(embeds, pos_table, proj_w, ln_gamma, ln_beta, seq_len)
    return h.reshape(bsz, seq_len, HID)
